# baseline (speedup 1.0000x reference)
import functools

import jax
import jax.numpy as jnp
from jax import lax
from jax.experimental import pallas as pl
from jax.experimental.pallas import tpu as pltpu
from jax.experimental.pallas import tpu_sc as plsc

N_NODES = 10000
NPAD = 10240
D = 128
DH2 = 64
NC, NS, L = 2, 16, 16
NW = NC * NS
K = 256
EPT = 20480
CH = EPT // K
EPAD = NS * EPT
CHD = EPAD // (NW * K)
RPT = NPAD // NS
PAD_NODE = N_NODES + 64
EPTW = CHD * K
CAPW = EPTW + 2 * K
CHW = CAPW // K
NMC, NEC, NNC = 1500, 500, 2500

_mesh = plsc.VectorSubcoreMesh(core_axis_name="c", subcore_axis_name="s",
                               num_cores=NC)


@functools.partial(
    pl.kernel,
    out_type=[
        jax.ShapeDtypeStruct((NC, 2, NPAD, 16), jnp.float32),
        jax.ShapeDtypeStruct((NW, CAPW), jnp.int32),
        jax.ShapeDtypeStruct((NW, CAPW), jnp.int32),
        jax.ShapeDtypeStruct((NW, CAPW), jnp.int32),
        jax.ShapeDtypeStruct((NW, 16), jnp.int32),
    ],
    mesh=_mesh,
    scratch_types=[
        pltpu.VMEM((CHD, K), jnp.int32),
        pltpu.VMEM((CHD, K), jnp.int32),
        pltpu.VMEM((K, 16), jnp.float32),
        pltpu.VMEM((CAPW,), jnp.int32),
        pltpu.VMEM((CAPW,), jnp.int32),
        pltpu.VMEM((CAPW,), jnp.int32),
        pltpu.VMEM((16,), jnp.int32),
        pltpu.VMEM_SHARED((NPAD, 16), jnp.float32),
        pltpu.VMEM_SHARED((NPAD, 16), jnp.float32),
        pltpu.SemaphoreType.DMA,
    ],
    compiler_params=pltpu.CompilerParams(use_tc_tiling_on_sc=False,
                                         needs_layout_passes=False),
)
def _sc_degrees(src_hbm, dst_hbm, ones_hbm, zeros_hbm,
                out_hbm, clo_hbm, chi_hbm, cdst_hbm, cnt_hbm,
                sidx, didx, ones_v, clo, chi, cdst, cntv, dgo_s, dgi_s, ssem):
    c = lax.axis_index("c")
    s = lax.axis_index("s")
    wid = c * NS + s
    pltpu.sync_copy(ones_hbm, ones_v)
    pltpu.sync_copy(zeros_hbm.at[pl.ds(s * RPT, RPT)], dgo_s.at[pl.ds(s * RPT, RPT)])
    pltpu.sync_copy(zeros_hbm.at[pl.ds(s * RPT, RPT)], dgi_s.at[pl.ds(s * RPT, RPT)])
    pltpu.sync_copy(src_hbm.at[wid], sidx)
    pltpu.sync_copy(dst_hbm.at[wid], didx)
    plsc.subcore_barrier()

    def eloop(t, carry):
        @pl.when(t >= 2)
        def _():
            pltpu.make_async_copy(ones_v, dgo_s.at[sidx.at[0]], ssem).wait()
            pltpu.make_async_copy(ones_v, dgi_s.at[didx.at[0]], ssem).wait()

        pltpu.async_copy(ones_v, dgo_s.at[sidx.at[t]], ssem, add=True)
        pltpu.async_copy(ones_v, dgi_s.at[didx.at[t]], ssem, add=True)
        return carry

    lax.fori_loop(0, CHD, eloop, 0)

    VPC = K // L

    def cloop(i, off):
        row = i // VPC
        col = (i % VPC) * L
        d = didx[row, pl.ds(col, L)]
        sv = sidx[row, pl.ds(col, L)]
        r = d - (d // NNC) * NNC
        mask = jnp.logical_and(r >= NMC, r < NMC + NEC)
        plsc.store_compressed(cdst.at[pl.ds(off, L)], d, mask=mask)
        plsc.store_compressed(clo.at[pl.ds(off, L)], sv, mask=mask)
        plsc.store_compressed(chi.at[pl.ds(off, L)], sv + NPAD, mask=mask)
        return off + jnp.sum(jnp.where(mask, 1, 0))

    n = lax.fori_loop(0, CHD * VPC, cloop, 0)
    bnd = ((n + 2 * K - 1) // (2 * K)) * (2 * K)
    padv = jnp.full((L,), PAD_NODE, jnp.int32)
    p0 = (n // L) * L
    rem = n - p0
    lanes = lax.iota(jnp.int32, L)
    keep = lanes < rem
    cdst[pl.ds(p0, L)] = jnp.where(keep, cdst[pl.ds(p0, L)], padv)
    clo[pl.ds(p0, L)] = jnp.where(keep, clo[pl.ds(p0, L)], padv)
    chi[pl.ds(p0, L)] = jnp.where(keep, chi[pl.ds(p0, L)], padv)

    def ploop(i, carry):
        base = p0 + (i + 1) * L
        cdst[pl.ds(base, L)] = padv
        clo[pl.ds(base, L)] = padv
        chi[pl.ds(base, L)] = padv
        return carry

    lax.fori_loop(0, jnp.maximum((bnd - p0) // L - 1, 0), ploop, 0)
    cntv[...] = jnp.zeros((L,), jnp.int32) + n
    pltpu.sync_copy(clo, clo_hbm.at[wid])
    pltpu.sync_copy(chi, chi_hbm.at[wid])
    pltpu.sync_copy(cdst, cdst_hbm.at[wid])
    pltpu.sync_copy(cntv, cnt_hbm.at[wid])

    for _ in range(2):
        pltpu.make_async_copy(ones_v, dgo_s.at[sidx.at[0]], ssem).wait()
        pltpu.make_async_copy(ones_v, dgi_s.at[didx.at[0]], ssem).wait()
    plsc.subcore_barrier()
    pltpu.sync_copy(dgo_s.at[pl.ds(s * RPT, RPT)],
                    out_hbm.at[c, 0, pl.ds(s * RPT, RPT)])
    pltpu.sync_copy(dgi_s.at[pl.ds(s * RPT, RPT)],
                    out_hbm.at[c, 1, pl.ds(s * RPT, RPT)])


@functools.partial(
    pl.kernel,
    out_type=jax.ShapeDtypeStruct((NC, NPAD, DH2), jnp.float32),
    mesh=_mesh,
    scratch_types=[
        pltpu.VMEM((CHW, K), jnp.int32),
        pltpu.VMEM((CHW, K), jnp.int32),
        pltpu.VMEM((2, K, DH2), jnp.float32),
        pltpu.VMEM((16,), jnp.int32),
        pltpu.VMEM_SHARED((NPAD, DH2), jnp.float32),
        pltpu.SemaphoreType.DMA,
        pltpu.SemaphoreType.DMA,
    ],
    compiler_params=pltpu.CompilerParams(use_tc_tiling_on_sc=False,
                                         needs_layout_passes=False),
)
def _sc_gs_entity(h_hbm, clo_hbm, chi_hbm, cdst_hbm, cnt_hbm, zeros_hbm, out_hbm,
                  sbuf, dbuf, rows_v, cntv, agg_s, gsem, ssem):
    c = lax.axis_index("c")
    s = lax.axis_index("s")
    pltpu.sync_copy(zeros_hbm.at[pl.ds(s * RPT, RPT)], agg_s.at[pl.ds(s * RPT, RPT)])
    plsc.subcore_barrier()

    for j in range(2):
        w = 2 * s + j
        pltpu.sync_copy(cnt_hbm.at[w], cntv)
        n = jnp.max(cntv[...])
        nch = 2 * ((n + 2 * K - 1) // (2 * K))

        @pl.when(nch > 0)
        def _():
            @pl.when(c == 0)
            def _():
                pltpu.sync_copy(clo_hbm.at[w], sbuf)

            @pl.when(c == 1)
            def _():
                pltpu.sync_copy(chi_hbm.at[w], sbuf)

            pltpu.sync_copy(cdst_hbm.at[w], dbuf)
            pltpu.async_copy(h_hbm.at[sbuf.at[0]], rows_v.at[0], gsem)

            def eloop(p, carry):
                for k in range(2):
                    t = 2 * p + k
                    pltpu.make_async_copy(
                        h_hbm.at[sbuf.at[t]], rows_v.at[k], gsem).wait()

                    @pl.when(t + 1 < nch)
                    def _():
                        @pl.when(t >= 1)
                        def _():
                            pltpu.make_async_copy(
                                rows_v.at[1 - k], agg_s.at[dbuf.at[0]], ssem).wait()

                        pltpu.async_copy(
                            h_hbm.at[sbuf.at[t + 1]], rows_v.at[1 - k], gsem)

                    pltpu.async_copy(rows_v.at[k], agg_s.at[dbuf.at[t]],
                                     ssem, add=True)
                return carry

            lax.fori_loop(0, nch // 2, eloop, 0)
            for jj in range(2):
                pltpu.make_async_copy(rows_v.at[jj], agg_s.at[dbuf.at[0]],
                                      ssem).wait()

    plsc.subcore_barrier()
    pltpu.sync_copy(agg_s.at[pl.ds(s * RPT, RPT)],
                    out_hbm.at[c, pl.ds(s * RPT, RPT)])


@functools.partial(
    pl.kernel,
    out_type=jax.ShapeDtypeStruct((NC, NPAD, DH2), jnp.float32),
    mesh=_mesh,
    scratch_types=[
        pltpu.VMEM((CH, K), jnp.int32),
        pltpu.VMEM((CH, K), jnp.int32),
        pltpu.VMEM((2, K, DH2), jnp.float32),
        pltpu.VMEM_SHARED((NPAD, DH2), jnp.float32),
        pltpu.SemaphoreType.DMA,
        pltpu.SemaphoreType.DMA,
    ],
    compiler_params=pltpu.CompilerParams(use_tc_tiling_on_sc=False),
)
def _sc_gather_scatter(h_hbm, src_hbm, dst_hbm, zeros_hbm, out_hbm,
                       sidx, didx, rows_v, agg_s, gsem, ssem):
    c = lax.axis_index("c")
    s = lax.axis_index("s")
    pltpu.sync_copy(zeros_hbm.at[pl.ds(s * RPT, RPT)], agg_s.at[pl.ds(s * RPT, RPT)])
    pltpu.sync_copy(src_hbm.at[c, s], sidx)
    pltpu.sync_copy(dst_hbm.at[s], didx)
    plsc.subcore_barrier()

    pltpu.async_copy(h_hbm.at[sidx.at[0]], rows_v.at[0], gsem)

    def eloop(t2, carry):
        for k in range(2):
            t = 2 * t2 + k
            pltpu.make_async_copy(h_hbm.at[sidx.at[t]], rows_v.at[k], gsem).wait()

            @pl.when(t + 1 < CH)
            def _():
                @pl.when(t >= 1)
                def _():
                    pltpu.make_async_copy(
                        rows_v.at[1 - k], agg_s.at[didx.at[0]], ssem).wait()

                pltpu.async_copy(h_hbm.at[sidx.at[t + 1]], rows_v.at[1 - k], gsem)

            pltpu.async_copy(rows_v.at[k], agg_s.at[didx.at[t]], ssem, add=True)
        return carry

    lax.fori_loop(0, CH // 2, eloop, 0)
    for j in range(2):
        pltpu.make_async_copy(rows_v.at[j], agg_s.at[didx.at[0]], ssem).wait()
    plsc.subcore_barrier()
    pltpu.sync_copy(agg_s.at[pl.ds(s * RPT, RPT)],
                    out_hbm.at[c, pl.ds(s * RPT, RPT)])


def _prep_body(dp_ref, node_ref, g_ref, b_ref, h_ref, norms_ref):
    do = dp_ref[0, 0] + dp_ref[1, 0]
    di = dp_ref[0, 1] + dp_ref[1, 1]
    nsb = lax.rsqrt(jnp.maximum(do, 1.0))
    norms_ref[0] = nsb
    norms_ref[1] = lax.rsqrt(jnp.maximum(di, 1.0))
    xb = node_ref[...]
    m = jnp.mean(xb, axis=1, keepdims=True)
    v = jnp.mean((xb - m) * (xb - m), axis=1, keepdims=True)
    y = ((xb - m) * lax.rsqrt(v + 1e-5) * g_ref[...] + b_ref[...]) * nsb
    h_ref[0] = y[:, :DH2]
    h_ref[1] = y[:, DH2:]


def _layer_body(agg_ref, nd_ref, ns_ref, w_ref, b_ref, o_ref):
    a = jnp.concatenate([agg_ref[0], agg_ref[1]], axis=1) * nd_ref[...]
    z = jnp.dot(a, w_ref[...], preferred_element_type=jnp.float32) + b_ref[...]
    z = jnp.where(z >= 0, z, 0.01 * z)
    y = z * ns_ref[...]
    o_ref[0] = y[:, :DH2]
    o_ref[1] = y[:, DH2:]


def _final_body(agg_ref, nd_ref, w_ref, b_ref, fw_ref, fb_ref, o_ref):
    a = jnp.concatenate([agg_ref[0], agg_ref[1]], axis=1) * nd_ref[...]
    z = jnp.dot(a, w_ref[...], preferred_element_type=jnp.float32) + b_ref[...]
    z = jnp.where(z >= 0, z, 0.01 * z)
    o_ref[...] = jnp.dot(z, fw_ref[...], preferred_element_type=jnp.float32) + fb_ref[...]


_R = 1024


def _tc_prep(degp4, node, g2, b2):
    return pl.pallas_call(
        _prep_body,
        grid=(NPAD // _R,),
        in_specs=[
            pl.BlockSpec((2, 2, _R, 1), lambda i: (0, 0, i, 0)),
            pl.BlockSpec((_R, D), lambda i: (i, 0)),
            pl.BlockSpec((1, D), lambda i: (0, 0)),
            pl.BlockSpec((1, D), lambda i: (0, 0)),
        ],
        out_specs=[
            pl.BlockSpec((2, _R, DH2), lambda i: (0, i, 0)),
            pl.BlockSpec((2, _R, 1), lambda i: (0, i, 0)),
        ],
        out_shape=[
            jax.ShapeDtypeStruct((2, NPAD, DH2), jnp.float32),
            jax.ShapeDtypeStruct((2, NPAD, 1), jnp.float32),
        ],
    )(degp4, node, g2, b2)


def _tc_layer(agg, nd, ns, W, b2):
    return pl.pallas_call(
        _layer_body,
        grid=(NPAD // _R,),
        in_specs=[
            pl.BlockSpec((2, _R, DH2), lambda i: (0, i, 0)),
            pl.BlockSpec((_R, 1), lambda i: (i, 0)),
            pl.BlockSpec((_R, 1), lambda i: (i, 0)),
            pl.BlockSpec((D, D), lambda i: (0, 0)),
            pl.BlockSpec((1, D), lambda i: (0, 0)),
        ],
        out_specs=pl.BlockSpec((2, _R, DH2), lambda i: (0, i, 0)),
        out_shape=jax.ShapeDtypeStruct((2, NPAD, DH2), jnp.float32),
    )(agg, nd, ns, W, b2)


def _tc_final(agg_e, nd_e, W, b2, fW, fb2):
    ne = agg_e.shape[1]
    return pl.pallas_call(
        _final_body,
        out_shape=jax.ShapeDtypeStruct((ne, D), jnp.float32),
    )(agg_e, nd_e, W, b2, fW, fb2)


def kernel(mention_hidden_state, entity_hidden_state, sent_hidden_state,
           edge_index, type_emb, ln_gamma, ln_beta, W0, b0, W1, b1, fc_W, fc_b):
    B, NM, DHS = mention_hidden_state.shape
    NE = entity_hidden_state.shape[1]
    NSn = sent_hidden_state.shape[1]
    DT = type_emb.shape[1]
    num_node = NM + NE + NSn
    E = edge_index.shape[1]

    m = jnp.concatenate(
        [mention_hidden_state,
         jnp.broadcast_to(type_emb[0].reshape(1, 1, DT), (B, NM, DT))], axis=2)
    e = jnp.concatenate(
        [entity_hidden_state,
         jnp.broadcast_to(type_emb[1].reshape(1, 1, DT), (B, NE, DT))], axis=2)
    sn = jnp.concatenate(
        [sent_hidden_state,
         jnp.broadcast_to(type_emb[2].reshape(1, 1, DT), (B, NSn, DT))], axis=2)
    node = jnp.concatenate((m, e, sn), axis=1).reshape(B * num_node, D)
    node = jnp.concatenate(
        [node, jnp.zeros((NPAD - B * num_node, D), jnp.float32)], axis=0)

    src = edge_index[0].astype(jnp.int32)
    dst = edge_index[1].astype(jnp.int32)
    pad_idx = jnp.full((EPAD - E,), PAD_NODE, jnp.int32)
    srcp = jnp.concatenate([src, pad_idx])
    dstp = jnp.concatenate([dst, pad_idx])
    src3d = srcp.reshape(NW, CHD, K)
    dst3d = dstp.reshape(NW, CHD, K)
    src4 = jnp.stack([srcp, srcp + NPAD]).reshape(NC, NS, CH, K)
    dst3 = dstp.reshape(NS, CH, K)

    ones16 = jnp.ones((K, 16), jnp.float32)
    zeros16 = jnp.zeros((NPAD, 16), jnp.float32)
    zeros64 = jnp.zeros((NPAD, DH2), jnp.float32)

    degp, cloL, chiL, cdstL, cnts = _sc_degrees(src3d, dst3d, ones16, zeros16)
    clo3 = cloL.reshape(NW, CHW, K)
    chi3 = chiL.reshape(NW, CHW, K)
    cdst3 = cdstL.reshape(NW, CHW, K)
    degp4 = degp[:, :, :, 0:1]
    g2 = ln_gamma.reshape(1, D)
    be2 = ln_beta.reshape(1, D)
    h0, norms = _tc_prep(degp4, node, g2, be2)
    ns = norms[0]
    nd = norms[1]

    agg1 = _sc_gather_scatter(h0.reshape(2 * NPAD, DH2), src4, dst3, zeros64)
    h1 = _tc_layer(agg1, nd, ns, W0, b0.reshape(1, D))
    agg2 = _sc_gs_entity(h1.reshape(2 * NPAD, DH2), clo3, chi3, cdst3, cnts, zeros64)

    agg2e = (agg2[:, :B * num_node]
             .reshape(2, B, num_node, DH2)[:, :, NM:NM + NE]
             .reshape(2, B * NE, DH2))
    nde = (nd[:B * num_node]
           .reshape(B, num_node, 1)[:, NM:NM + NE]
           .reshape(B * NE, 1))
    out = _tc_final(agg2e, nde, W1, b1.reshape(1, D),
                    fc_W, fc_b.reshape(1, D))
    return out.reshape(B, NE, D)


# speedup vs baseline: 1.0490x; 1.0490x over previous
import functools

import jax
import jax.numpy as jnp
from jax import lax
from jax.experimental import pallas as pl
from jax.experimental.pallas import tpu as pltpu
from jax.experimental.pallas import tpu_sc as plsc

N_NODES = 10000
NPAD = 10240
D = 128
DH2 = 64
NC, NS, L = 2, 16, 16
NW = NC * NS
K = 256
EPT = 20480
CH = EPT // K
EPAD = NS * EPT
CHD = EPAD // (NW * K)
RPT = NPAD // NS
PAD_NODE = N_NODES + 64
EPTW = CHD * K
CAPW = EPTW + 2 * K
CHW = CAPW // K
NMC, NEC, NNC = 1500, 500, 2500

_mesh = plsc.VectorSubcoreMesh(core_axis_name="c", subcore_axis_name="s",
                               num_cores=NC)


@functools.partial(
    pl.kernel,
    out_type=[
        jax.ShapeDtypeStruct((NC, 2, NPAD, 16), jnp.float32),
        jax.ShapeDtypeStruct((NW, CAPW), jnp.int32),
        jax.ShapeDtypeStruct((NW, CAPW), jnp.int32),
        jax.ShapeDtypeStruct((NW, CAPW), jnp.int32),
        jax.ShapeDtypeStruct((NW, 16), jnp.int32),
    ],
    mesh=_mesh,
    scratch_types=[
        pltpu.VMEM((CHD, K), jnp.int32),
        pltpu.VMEM((CHD, K), jnp.int32),
        pltpu.VMEM((K, 16), jnp.float32),
        pltpu.VMEM((CAPW,), jnp.int32),
        pltpu.VMEM((CAPW,), jnp.int32),
        pltpu.VMEM((CAPW,), jnp.int32),
        pltpu.VMEM((16,), jnp.int32),
        pltpu.VMEM_SHARED((NPAD, 16), jnp.float32),
        pltpu.VMEM_SHARED((NPAD, 16), jnp.float32),
        pltpu.SemaphoreType.DMA,
    ],
    compiler_params=pltpu.CompilerParams(use_tc_tiling_on_sc=False,
                                         needs_layout_passes=False),
)
def _sc_degrees(src_hbm, dst_hbm, ones_hbm, zeros_hbm,
                out_hbm, clo_hbm, chi_hbm, cdst_hbm, cnt_hbm,
                sidx, didx, ones_v, clo, chi, cdst, cntv, dgo_s, dgi_s, ssem):
    c = lax.axis_index("c")
    s = lax.axis_index("s")
    wid = c * NS + s
    pltpu.sync_copy(ones_hbm, ones_v)
    pltpu.sync_copy(zeros_hbm.at[pl.ds(s * RPT, RPT)], dgo_s.at[pl.ds(s * RPT, RPT)])
    pltpu.sync_copy(zeros_hbm.at[pl.ds(s * RPT, RPT)], dgi_s.at[pl.ds(s * RPT, RPT)])
    pltpu.sync_copy(src_hbm.at[wid], sidx)
    pltpu.sync_copy(dst_hbm.at[wid], didx)
    plsc.subcore_barrier()

    def eloop(t, carry):
        @pl.when(t >= 2)
        def _():
            pltpu.make_async_copy(ones_v, dgo_s.at[sidx.at[0]], ssem).wait()
            pltpu.make_async_copy(ones_v, dgi_s.at[didx.at[0]], ssem).wait()

        pltpu.async_copy(ones_v, dgo_s.at[sidx.at[t]], ssem, add=True)
        pltpu.async_copy(ones_v, dgi_s.at[didx.at[t]], ssem, add=True)
        return carry

    lax.fori_loop(0, CHD, eloop, 0)

    VPC = K // L

    def cloop(i, off):
        row = i // VPC
        col = (i % VPC) * L
        d = didx[row, pl.ds(col, L)]
        sv = sidx[row, pl.ds(col, L)]
        r = d - (d // NNC) * NNC
        mask = jnp.logical_and(r >= NMC, r < NMC + NEC)
        plsc.store_compressed(cdst.at[pl.ds(off, L)], d, mask=mask)
        plsc.store_compressed(clo.at[pl.ds(off, L)], sv, mask=mask)
        plsc.store_compressed(chi.at[pl.ds(off, L)], sv + NPAD, mask=mask)
        return off + jnp.sum(jnp.where(mask, 1, 0))

    n = lax.fori_loop(0, CHD * VPC, cloop, 0)
    bnd = ((n + 2 * K - 1) // (2 * K)) * (2 * K)
    padv = jnp.full((L,), PAD_NODE, jnp.int32)
    p0 = (n // L) * L
    rem = n - p0
    lanes = lax.iota(jnp.int32, L)
    keep = lanes < rem
    cdst[pl.ds(p0, L)] = jnp.where(keep, cdst[pl.ds(p0, L)], padv)
    clo[pl.ds(p0, L)] = jnp.where(keep, clo[pl.ds(p0, L)], padv)
    chi[pl.ds(p0, L)] = jnp.where(keep, chi[pl.ds(p0, L)], padv)

    def ploop(i, carry):
        base = p0 + (i + 1) * L
        cdst[pl.ds(base, L)] = padv
        clo[pl.ds(base, L)] = padv
        chi[pl.ds(base, L)] = padv
        return carry

    lax.fori_loop(0, jnp.maximum((bnd - p0) // L - 1, 0), ploop, 0)
    cntv[...] = jnp.zeros((L,), jnp.int32) + n
    pltpu.sync_copy(clo, clo_hbm.at[wid])
    pltpu.sync_copy(chi, chi_hbm.at[wid])
    pltpu.sync_copy(cdst, cdst_hbm.at[wid])
    pltpu.sync_copy(cntv, cnt_hbm.at[wid])

    for _ in range(2):
        pltpu.make_async_copy(ones_v, dgo_s.at[sidx.at[0]], ssem).wait()
        pltpu.make_async_copy(ones_v, dgi_s.at[didx.at[0]], ssem).wait()
    plsc.subcore_barrier()
    pltpu.sync_copy(dgo_s.at[pl.ds(s * RPT, RPT)],
                    out_hbm.at[c, 0, pl.ds(s * RPT, RPT)])
    pltpu.sync_copy(dgi_s.at[pl.ds(s * RPT, RPT)],
                    out_hbm.at[c, 1, pl.ds(s * RPT, RPT)])


NENT = 4 * NEC
ENT_PT = NENT // NS


@functools.partial(
    pl.kernel,
    out_type=jax.ShapeDtypeStruct((NC, NENT, DH2), jnp.float32),
    mesh=_mesh,
    scratch_types=[
        pltpu.VMEM((CHW, K), jnp.int32),
        pltpu.VMEM((CHW, K), jnp.int32),
        pltpu.VMEM((2, K, DH2), jnp.float32),
        pltpu.VMEM((16,), jnp.int32),
        pltpu.VMEM_SHARED((NPAD, DH2), jnp.float32),
        pltpu.SemaphoreType.DMA,
        pltpu.SemaphoreType.DMA,
    ],
    compiler_params=pltpu.CompilerParams(use_tc_tiling_on_sc=False,
                                         needs_layout_passes=False),
)
def _sc_gs_entity(h_hbm, clo_hbm, chi_hbm, cdst_hbm, cnt_hbm, zeros_hbm, out_hbm,
                  sbuf, dbuf, rows_v, cntv, agg_s, gsem, ssem):
    c = lax.axis_index("c")
    s = lax.axis_index("s")
    row0 = (s // 4) * NNC + NMC + (s % 4) * ENT_PT
    pltpu.sync_copy(zeros_hbm.at[pl.ds(0, ENT_PT)], agg_s.at[pl.ds(row0, ENT_PT)])
    plsc.subcore_barrier()

    for j in range(2):
        w = 2 * s + j
        pltpu.sync_copy(cnt_hbm.at[w], cntv)
        n = jnp.max(cntv[...])
        nch = 2 * ((n + 2 * K - 1) // (2 * K))

        @pl.when(nch > 0)
        def _():
            @pl.when(c == 0)
            def _():
                pltpu.sync_copy(clo_hbm.at[w], sbuf)

            @pl.when(c == 1)
            def _():
                pltpu.sync_copy(chi_hbm.at[w], sbuf)

            pltpu.sync_copy(cdst_hbm.at[w], dbuf)
            pltpu.async_copy(h_hbm.at[sbuf.at[0]], rows_v.at[0], gsem)

            def eloop(p, carry):
                for k in range(2):
                    t = 2 * p + k
                    pltpu.make_async_copy(
                        h_hbm.at[sbuf.at[t]], rows_v.at[k], gsem).wait()

                    @pl.when(t + 1 < nch)
                    def _():
                        @pl.when(t >= 1)
                        def _():
                            pltpu.make_async_copy(
                                rows_v.at[1 - k], agg_s.at[dbuf.at[0]], ssem).wait()

                        pltpu.async_copy(
                            h_hbm.at[sbuf.at[t + 1]], rows_v.at[1 - k], gsem)

                    pltpu.async_copy(rows_v.at[k], agg_s.at[dbuf.at[t]],
                                     ssem, add=True)
                return carry

            lax.fori_loop(0, nch // 2, eloop, 0)
            for jj in range(2):
                pltpu.make_async_copy(rows_v.at[jj], agg_s.at[dbuf.at[0]],
                                      ssem).wait()

    plsc.subcore_barrier()
    pltpu.sync_copy(agg_s.at[pl.ds(row0, ENT_PT)],
                    out_hbm.at[c, pl.ds(s * ENT_PT, ENT_PT)])


@functools.partial(
    pl.kernel,
    out_type=jax.ShapeDtypeStruct((NC, NPAD, DH2), jnp.float32),
    mesh=_mesh,
    scratch_types=[
        pltpu.VMEM((CH, K), jnp.int32),
        pltpu.VMEM((CH, K), jnp.int32),
        pltpu.VMEM((2, K, DH2), jnp.float32),
        pltpu.VMEM_SHARED((NPAD, DH2), jnp.float32),
        pltpu.SemaphoreType.DMA,
        pltpu.SemaphoreType.DMA,
    ],
    compiler_params=pltpu.CompilerParams(use_tc_tiling_on_sc=False),
)
def _sc_gather_scatter(h_hbm, src_hbm, dst_hbm, zeros_hbm, out_hbm,
                       sidx, didx, rows_v, agg_s, gsem, ssem):
    c = lax.axis_index("c")
    s = lax.axis_index("s")
    pltpu.sync_copy(zeros_hbm.at[pl.ds(s * RPT, RPT)], agg_s.at[pl.ds(s * RPT, RPT)])
    pltpu.sync_copy(src_hbm.at[c, s], sidx)
    pltpu.sync_copy(dst_hbm.at[s], didx)
    plsc.subcore_barrier()

    pltpu.async_copy(h_hbm.at[sidx.at[0]], rows_v.at[0], gsem)

    def eloop(t2, carry):
        for k in range(2):
            t = 2 * t2 + k
            pltpu.make_async_copy(h_hbm.at[sidx.at[t]], rows_v.at[k], gsem).wait()

            @pl.when(t + 1 < CH)
            def _():
                @pl.when(t >= 1)
                def _():
                    pltpu.make_async_copy(
                        rows_v.at[1 - k], agg_s.at[didx.at[0]], ssem).wait()

                pltpu.async_copy(h_hbm.at[sidx.at[t + 1]], rows_v.at[1 - k], gsem)

            pltpu.async_copy(rows_v.at[k], agg_s.at[didx.at[t]], ssem, add=True)
        return carry

    lax.fori_loop(0, CH // 2, eloop, 0)
    for j in range(2):
        pltpu.make_async_copy(rows_v.at[j], agg_s.at[didx.at[0]], ssem).wait()
    plsc.subcore_barrier()
    pltpu.sync_copy(agg_s.at[pl.ds(s * RPT, RPT)],
                    out_hbm.at[c, pl.ds(s * RPT, RPT)])


def _prep_body(dp_ref, node_ref, g_ref, b_ref, h_ref, norms_ref):
    do = dp_ref[0, 0] + dp_ref[1, 0]
    di = dp_ref[0, 1] + dp_ref[1, 1]
    nsb = lax.rsqrt(jnp.maximum(do, 1.0))
    norms_ref[0] = nsb
    norms_ref[1] = lax.rsqrt(jnp.maximum(di, 1.0))
    xb = node_ref[...]
    m = jnp.mean(xb, axis=1, keepdims=True)
    v = jnp.mean((xb - m) * (xb - m), axis=1, keepdims=True)
    y = ((xb - m) * lax.rsqrt(v + 1e-5) * g_ref[...] + b_ref[...]) * nsb
    h_ref[0] = y[:, :DH2]
    h_ref[1] = y[:, DH2:]


def _layer_body(agg_ref, nd_ref, ns_ref, w_ref, b_ref, o_ref):
    a = jnp.concatenate([agg_ref[0], agg_ref[1]], axis=1) * nd_ref[...]
    z = jnp.dot(a, w_ref[...], preferred_element_type=jnp.float32) + b_ref[...]
    z = jnp.where(z >= 0, z, 0.01 * z)
    y = z * ns_ref[...]
    o_ref[0] = y[:, :DH2]
    o_ref[1] = y[:, DH2:]


def _final_body(agg_ref, nd_ref, w_ref, b_ref, fw_ref, fb_ref, o_ref):
    a = jnp.concatenate([agg_ref[0], agg_ref[1]], axis=1) * nd_ref[...]
    z = jnp.dot(a, w_ref[...], preferred_element_type=jnp.float32) + b_ref[...]
    z = jnp.where(z >= 0, z, 0.01 * z)
    o_ref[...] = jnp.dot(z, fw_ref[...], preferred_element_type=jnp.float32) + fb_ref[...]


_R = 1024


def _tc_prep(degp4, node, g2, b2):
    return pl.pallas_call(
        _prep_body,
        grid=(NPAD // _R,),
        in_specs=[
            pl.BlockSpec((2, 2, _R, 1), lambda i: (0, 0, i, 0)),
            pl.BlockSpec((_R, D), lambda i: (i, 0)),
            pl.BlockSpec((1, D), lambda i: (0, 0)),
            pl.BlockSpec((1, D), lambda i: (0, 0)),
        ],
        out_specs=[
            pl.BlockSpec((2, _R, DH2), lambda i: (0, i, 0)),
            pl.BlockSpec((2, _R, 1), lambda i: (0, i, 0)),
        ],
        out_shape=[
            jax.ShapeDtypeStruct((2, NPAD, DH2), jnp.float32),
            jax.ShapeDtypeStruct((2, NPAD, 1), jnp.float32),
        ],
    )(degp4, node, g2, b2)


def _tc_layer(agg, nd, ns, W, b2):
    return pl.pallas_call(
        _layer_body,
        grid=(NPAD // _R,),
        in_specs=[
            pl.BlockSpec((2, _R, DH2), lambda i: (0, i, 0)),
            pl.BlockSpec((_R, 1), lambda i: (i, 0)),
            pl.BlockSpec((_R, 1), lambda i: (i, 0)),
            pl.BlockSpec((D, D), lambda i: (0, 0)),
            pl.BlockSpec((1, D), lambda i: (0, 0)),
        ],
        out_specs=pl.BlockSpec((2, _R, DH2), lambda i: (0, i, 0)),
        out_shape=jax.ShapeDtypeStruct((2, NPAD, DH2), jnp.float32),
    )(agg, nd, ns, W, b2)


def _tc_final(agg_e, nd_e, W, b2, fW, fb2):
    ne = agg_e.shape[1]
    return pl.pallas_call(
        _final_body,
        out_shape=jax.ShapeDtypeStruct((ne, D), jnp.float32),
    )(agg_e, nd_e, W, b2, fW, fb2)


def kernel(mention_hidden_state, entity_hidden_state, sent_hidden_state,
           edge_index, type_emb, ln_gamma, ln_beta, W0, b0, W1, b1, fc_W, fc_b):
    B, NM, DHS = mention_hidden_state.shape
    NE = entity_hidden_state.shape[1]
    NSn = sent_hidden_state.shape[1]
    DT = type_emb.shape[1]
    num_node = NM + NE + NSn
    E = edge_index.shape[1]

    m = jnp.concatenate(
        [mention_hidden_state,
         jnp.broadcast_to(type_emb[0].reshape(1, 1, DT), (B, NM, DT))], axis=2)
    e = jnp.concatenate(
        [entity_hidden_state,
         jnp.broadcast_to(type_emb[1].reshape(1, 1, DT), (B, NE, DT))], axis=2)
    sn = jnp.concatenate(
        [sent_hidden_state,
         jnp.broadcast_to(type_emb[2].reshape(1, 1, DT), (B, NSn, DT))], axis=2)
    node = jnp.concatenate((m, e, sn), axis=1).reshape(B * num_node, D)
    node = jnp.concatenate(
        [node, jnp.zeros((NPAD - B * num_node, D), jnp.float32)], axis=0)

    src = edge_index[0].astype(jnp.int32)
    dst = edge_index[1].astype(jnp.int32)
    pad_idx = jnp.full((EPAD - E,), PAD_NODE, jnp.int32)
    srcp = jnp.concatenate([src, pad_idx])
    dstp = jnp.concatenate([dst, pad_idx])
    src3d = srcp.reshape(NW, CHD, K)
    dst3d = dstp.reshape(NW, CHD, K)
    src4 = jnp.stack([srcp, srcp + NPAD]).reshape(NC, NS, CH, K)
    dst3 = dstp.reshape(NS, CH, K)

    ones16 = jnp.ones((K, 16), jnp.float32)
    zeros16 = jnp.zeros((NPAD, 16), jnp.float32)
    zeros64 = jnp.zeros((NPAD, DH2), jnp.float32)

    degp, cloL, chiL, cdstL, cnts = _sc_degrees(src3d, dst3d, ones16, zeros16)
    clo3 = cloL.reshape(NW, CHW, K)
    chi3 = chiL.reshape(NW, CHW, K)
    cdst3 = cdstL.reshape(NW, CHW, K)
    degp4 = degp[:, :, :, 0:1]
    g2 = ln_gamma.reshape(1, D)
    be2 = ln_beta.reshape(1, D)
    h0, norms = _tc_prep(degp4, node, g2, be2)
    ns = norms[0]
    nd = norms[1]

    agg1 = _sc_gather_scatter(h0.reshape(2 * NPAD, DH2), src4, dst3, zeros64)
    h1 = _tc_layer(agg1, nd, ns, W0, b0.reshape(1, D))
    agg2e = _sc_gs_entity(h1.reshape(2 * NPAD, DH2), clo3, chi3, cdst3, cnts,
                          zeros64)
    nde = (nd[:B * num_node]
           .reshape(B, num_node, 1)[:, NM:NM + NE]
           .reshape(B * NE, 1))
    out = _tc_final(agg2e, nde, W1, b1.reshape(1, D),
                    fc_W, fc_b.reshape(1, D))
    return out.reshape(B, NE, D)
